# baseline (speedup 1.0000x reference)
import functools

import jax
import jax.numpy as jnp
from jax import lax
from jax.experimental import pallas as pl
from jax.experimental.pallas import tpu as pltpu
from jax.experimental.pallas import tpu_sc as plsc

_N_CLS = 1000
_N_CTX = 16
_DIM = 512
_SEQ = 77
_SUF = _SEQ - 1 - _N_CTX
_ROW = _SEQ * _DIM
_CTX_LEN = _N_CTX * _DIM
_SUF_LEN = _SUF * _DIM
_SUF_OFF = _DIM + _CTX_LEN
_REP = 8
_SPAN = 32


@functools.cache
def _build_sc_kernel():
    info = plsc.get_sparse_core_info()
    nc, ns = info.num_cores, info.num_subcores
    nw = nc * ns
    n_main = nw - 1
    tail_start = n_main * _SPAN
    tail_span = _N_CLS - tail_start

    mesh = plsc.VectorSubcoreMesh(core_axis_name="c", subcore_axis_name="s")

    @functools.partial(
        pl.kernel,
        out_type=jax.ShapeDtypeStruct((_N_CLS, _ROW), jnp.float32),
        mesh=mesh,
        scratch_types=[
            pltpu.VMEM((_REP, _CTX_LEN), jnp.float32),
            pltpu.SemaphoreType.DMA,
        ],
    )
    def prompts_kernel(ctx_hbm, pre_hbm, suf_hbm, out_hbm, ctx_v, sem):
        wid = lax.axis_index("s") * nc + lax.axis_index("c")
        for r in range(_REP):
            pltpu.sync_copy(ctx_hbm, ctx_v.at[r])

        def move_span(start, span):
            copies = [
                pltpu.make_async_copy(
                    suf_hbm.at[pl.ds(start, span)],
                    out_hbm.at[pl.ds(start, span), pl.ds(_SUF_OFF, _SUF_LEN)],
                    sem,
                ),
                pltpu.make_async_copy(
                    pre_hbm.at[pl.ds(start, span)],
                    out_hbm.at[pl.ds(start, span), pl.ds(0, _DIM)],
                    sem,
                ),
            ]
            for off in range(0, span, _REP):
                k = min(_REP, span - off)
                copies.append(pltpu.make_async_copy(
                    ctx_v.at[pl.ds(0, k)],
                    out_hbm.at[pl.ds(start + off, k), pl.ds(_DIM, _CTX_LEN)],
                    sem,
                ))
            for c in copies:
                c.start()
            for c in copies:
                c.wait()

        @pl.when(wid < n_main)
        def _():
            move_span(pl.multiple_of(wid * _SPAN, 8), _SPAN)

        @pl.when(wid == n_main)
        def _():
            move_span(tail_start, tail_span)

    return prompts_kernel


def kernel(ctx, token_prefix, token_suffix):
    out2d = _build_sc_kernel()(
        ctx.reshape(_CTX_LEN),
        token_prefix.reshape(_N_CLS, _DIM),
        token_suffix.reshape(_N_CLS, _SUF_LEN),
    )
    return out2d.reshape(_N_CLS, _SEQ, _DIM)


# speedup vs baseline: 7.4232x; 7.4232x over previous
import functools

import jax
import jax.numpy as jnp
from jax import lax
from jax.experimental import pallas as pl
from jax.experimental.pallas import tpu as pltpu
from jax.experimental.pallas import tpu_sc as plsc

_N_CLS = 1000
_N_CTX = 16
_DIM = 512
_SEQ = 77
_SUF = _SEQ - 1 - _N_CTX
_ROW = _SEQ * _DIM
_CTX_LEN = _N_CTX * _DIM
_SUF_LEN = _SUF * _DIM
_SUF_OFF = _DIM + _CTX_LEN
_NBUF = 3


@functools.cache
def _build_sc_kernel():
    info = plsc.get_sparse_core_info()
    nc, ns = info.num_cores, info.num_subcores
    nw = nc * ns
    base_cnt, extra = divmod(_N_CLS, nw)
    max_cnt = base_cnt + (1 if extra else 0)
    steps = max_cnt + (-max_cnt) % _NBUF

    mesh = plsc.VectorSubcoreMesh(core_axis_name="c", subcore_axis_name="s")

    @functools.partial(
        pl.kernel,
        out_type=jax.ShapeDtypeStruct((_N_CLS, _ROW), jnp.float32),
        mesh=mesh,
        scratch_types=[
            pltpu.VMEM((_ROW,), jnp.float32),
            pltpu.VMEM((_ROW,), jnp.float32),
            pltpu.VMEM((_ROW,), jnp.float32),
            pltpu.SemaphoreType.DMA((_NBUF,)),
            pltpu.SemaphoreType.DMA((_NBUF,)),
        ],
    )
    def prompts_kernel(ctx_hbm, pre_hbm, suf_hbm, out_hbm,
                       buf0, buf1, buf2, lsem, ssem):
        bufs = (buf0, buf1, buf2)
        wid = lax.axis_index("s") * nc + lax.axis_index("c")
        cnt = base_cnt + (wid < extra).astype(jnp.int32)
        start = wid * base_cnt + jnp.minimum(wid, extra)

        for b in range(_NBUF):
            pltpu.sync_copy(ctx_hbm, bufs[b].at[pl.ds(_DIM, _CTX_LEN)])

        def load(b, c):
            pltpu.make_async_copy(
                pre_hbm.at[c], bufs[b].at[pl.ds(0, _DIM)], lsem.at[b]).start()
            pltpu.make_async_copy(
                suf_hbm.at[c], bufs[b].at[pl.ds(_SUF_OFF, _SUF_LEN)],
                lsem.at[b]).start()

        def wait_load(b):
            pltpu.make_async_copy(
                pre_hbm.at[0], bufs[b].at[pl.ds(0, _DIM)], lsem.at[b]).wait()
            pltpu.make_async_copy(
                suf_hbm.at[0], bufs[b].at[pl.ds(_SUF_OFF, _SUF_LEN)],
                lsem.at[b]).wait()

        def store(b, c):
            pltpu.make_async_copy(bufs[b], out_hbm.at[c], ssem.at[b]).start()

        def wait_store(b):
            pltpu.make_async_copy(bufs[b], out_hbm.at[0], ssem.at[b]).wait()

        load(0, start)
        load(1, start + 1)

        for i in range(steps):
            b = i % _NBUF
            if i < max_cnt:
                @pl.when(i < cnt)
                def _(i=i, b=b):
                    wait_load(b)
                    store(b, start + i)
            if i + 2 < steps and i + 2 < max_cnt:
                nb = (i + 2) % _NBUF

                @pl.when(i + 2 < cnt)
                def _(i=i, nb=nb):
                    if i >= 1:
                        wait_store(nb)
                    load(nb, start + i + 2)

        for b in range(_NBUF):
            wait_store(b)

    return prompts_kernel


def kernel(ctx, token_prefix, token_suffix):
    out2d = _build_sc_kernel()(
        ctx.reshape(_CTX_LEN),
        token_prefix.reshape(_N_CLS, _DIM),
        token_suffix.reshape(_N_CLS, _SUF_LEN),
    )
    return out2d.reshape(_N_CLS, _SEQ, _DIM)


# speedup vs baseline: 31.5522x; 4.2505x over previous
import functools

import jax
import jax.numpy as jnp
from jax import lax
from jax.experimental import pallas as pl
from jax.experimental.pallas import tpu as pltpu
from jax.experimental.pallas import tpu_sc as plsc

_N_CLS = 1000
_N_CTX = 16
_DIM = 512
_SEQ = 77
_SUF = _SEQ - 1 - _N_CTX
_NCOPY = 1 + _SUF
_CH = 40
_NCHUNK = _N_CLS // _CH
_TOTAL = _NCOPY * _NCHUNK
_NBUF = 3

_N_A = 54
_N_B = 41
_TAIL = _TOTAL - 16 * _N_A - 16 * _N_B


@functools.cache
def _build_sc_kernel():
    info = plsc.get_sparse_core_info()
    nc, ns = info.num_cores, info.num_subcores
    mesh = plsc.VectorSubcoreMesh(core_axis_name="c", subcore_axis_name="s")

    @functools.partial(
        pl.kernel,
        out_type=jax.ShapeDtypeStruct((_SEQ, _N_CLS, _DIM), jnp.float32),
        mesh=mesh,
        scratch_types=[
            pltpu.VMEM((_CH, _DIM), jnp.float32),
            pltpu.VMEM((_CH, _DIM), jnp.float32),
            pltpu.VMEM((_CH, _DIM), jnp.float32),
            pltpu.VMEM((_CH, _DIM), jnp.float32),
            pltpu.SemaphoreType.DMA((_NBUF,)),
            pltpu.SemaphoreType.DMA((_NBUF,)),
            pltpu.SemaphoreType.DMA,
        ],
    )
    def prompts_kernel(ctx_hbm, pre_hbm, suf_hbm, out_hbm,
                       buf0, buf1, buf2, rep, lsem, ssem, csem):
        bufs = (buf0, buf1, buf2)
        wid = lax.axis_index("s") * nc + lax.axis_index("c")

        def chunk_coords(g):
            plane = lax.div(g, _NCHUNK)
            off = lax.rem(g, _NCHUNK) * _CH
            dst = jnp.where(plane == 0, 0, plane + _N_CTX)
            return plane, off, dst

        def load(b, g):
            plane, off, _ = chunk_coords(g)

            @pl.when(plane == 0)
            def _():
                pltpu.make_async_copy(
                    pre_hbm.at[pl.ds(off, _CH)], bufs[b], lsem.at[b]).start()

            @pl.when(plane != 0)
            def _():
                pltpu.make_async_copy(
                    suf_hbm.at[plane - 1, pl.ds(off, _CH)], bufs[b],
                    lsem.at[b]).start()

        def wait_load(b):
            pltpu.make_async_copy(
                pre_hbm.at[pl.ds(0, _CH)], bufs[b], lsem.at[b]).wait()

        def store(b, g):
            _, off, dst = chunk_coords(g)
            pltpu.make_async_copy(
                bufs[b], out_hbm.at[dst, pl.ds(off, _CH)], ssem.at[b]).start()

        def wait_store(b):
            pltpu.make_async_copy(
                bufs[b], out_hbm.at[0, pl.ds(0, _CH)], ssem.at[b]).wait()

        def run_pipeline(g0, n):
            load(0, g0)
            load(1, g0 + 1)
            for i in range(n):
                b = i % _NBUF
                wait_load(b)
                store(b, g0 + i)
                if i + 2 < n:
                    nb = (i + 2) % _NBUF
                    if i >= 1:
                        wait_store(nb)
                    load(nb, g0 + i + 2)
            for b in range(_NBUF):
                wait_store(b)

        def serial_chunk(g):
            load(0, g)
            wait_load(0)
            store(0, g)
            wait_store(0)

        @pl.when(wid < 16)
        def _():
            run_pipeline(wid * _N_A, _N_A)
            @pl.when(wid < _TAIL)
            def _():
                serial_chunk(16 * _N_A + 16 * _N_B + wid)

        @pl.when(wid >= 16)
        def _():
            r = wid - 16
            pltpu.sync_copy(ctx_hbm.at[r], rep.at[0])

            def fill(k, carry):
                pltpu.sync_copy(ctx_hbm.at[r], rep.at[k])
                return carry

            lax.fori_loop(1, _CH, fill, 0)
            for j in range(_NCHUNK):
                pltpu.make_async_copy(
                    rep, out_hbm.at[r + 1, pl.ds(j * _CH, _CH)], csem).start()
            run_pipeline(16 * _N_A + (wid - 16) * _N_B, _N_B)
            for j in range(_NCHUNK):
                pltpu.make_async_copy(
                    rep, out_hbm.at[1, pl.ds(0, _CH)], csem).wait()

    return prompts_kernel


def kernel(ctx, token_prefix, token_suffix):
    out_t = _build_sc_kernel()(
        ctx,
        token_prefix.reshape(_N_CLS, _DIM),
        token_suffix.transpose(1, 0, 2),
    )
    return out_t.transpose(1, 0, 2)


# speedup vs baseline: 33.3349x; 1.0565x over previous
import functools

import jax
import jax.numpy as jnp
from jax import lax
from jax.experimental import pallas as pl
from jax.experimental.pallas import tpu as pltpu
from jax.experimental.pallas import tpu_sc as plsc

_N_CLS = 1000
_N_CTX = 16
_DIM = 512
_SEQ = 77
_SUF = _SEQ - 1 - _N_CTX
_NCOPY = 1 + _SUF
_CH = 40
_NCHUNK = _N_CLS // _CH
_TOTAL = _NCOPY * _NCHUNK

_N_A = 54
_N_B = 41
_TAIL = _TOTAL - 16 * _N_A - 16 * _N_B
_NBUF = 5


@functools.cache
def _build_sc_kernel():
    info = plsc.get_sparse_core_info()
    nc, ns = info.num_cores, info.num_subcores
    mesh = plsc.VectorSubcoreMesh(core_axis_name="c", subcore_axis_name="s")

    @functools.partial(
        pl.kernel,
        out_type=jax.ShapeDtypeStruct((_SEQ, _N_CLS, _DIM), jnp.float32),
        mesh=mesh,
        scratch_types=(
            [pltpu.VMEM((_CH, _DIM), jnp.float32) for _ in range(_NBUF)]
            + [
                pltpu.VMEM((_CH, _DIM), jnp.float32),
                pltpu.SemaphoreType.DMA((_NBUF,)),
                pltpu.SemaphoreType.DMA((_NBUF,)),
                pltpu.SemaphoreType.DMA,
            ]
        ),
    )
    def prompts_kernel(ctx_hbm, pre_hbm, suf_hbm, out_hbm,
                       *scratch):
        bufs = scratch[:_NBUF]
        rep, lsem, ssem, csem = scratch[_NBUF:]
        wid = lax.axis_index("s") * nc + lax.axis_index("c")

        def chunk_coords(g):
            plane = lax.div(g, _NCHUNK)
            off = lax.rem(g, _NCHUNK) * _CH
            dst = jnp.where(plane == 0, 0, plane + _N_CTX)
            return plane, off, dst

        def load(b, g):
            plane, off, _ = chunk_coords(g)

            @pl.when(plane == 0)
            def _():
                pltpu.make_async_copy(
                    pre_hbm.at[pl.ds(off, _CH)], bufs[b], lsem.at[b]).start()

            @pl.when(plane != 0)
            def _():
                pltpu.make_async_copy(
                    suf_hbm.at[plane - 1, pl.ds(off, _CH)], bufs[b],
                    lsem.at[b]).start()

        def wait_load(b):
            pltpu.make_async_copy(
                pre_hbm.at[pl.ds(0, _CH)], bufs[b], lsem.at[b]).wait()

        def store(b, g):
            _, off, dst = chunk_coords(g)
            pltpu.make_async_copy(
                bufs[b], out_hbm.at[dst, pl.ds(off, _CH)], ssem.at[b]).start()

        def wait_store(b):
            pltpu.make_async_copy(
                bufs[b], out_hbm.at[0, pl.ds(0, _CH)], ssem.at[b]).wait()

        def run_pipeline(g0, n):
            assert n >= _NBUF + 1
            for j in range(_NBUF - 1):
                load(j, g0 + j)
            for i in range(n):
                b = i % _NBUF
                wait_load(b)
                store(b, g0 + i)
                if i + _NBUF - 1 < n:
                    nb = (i + _NBUF - 1) % _NBUF
                    if i >= 1:
                        wait_store(nb)
                    load(nb, g0 + i + _NBUF - 1)
            for b in range(_NBUF):
                wait_store(b)

        def serial_chunk(g):
            load(0, g)
            wait_load(0)
            store(0, g)
            wait_store(0)

        @pl.when(wid < 16)
        def _():
            run_pipeline(wid * _N_A, _N_A)
            @pl.when(wid < _TAIL)
            def _():
                serial_chunk(16 * _N_A + 16 * _N_B + wid)

        @pl.when(wid >= 16)
        def _():
            r = wid - 16
            def fill(k, carry):
                pltpu.make_async_copy(ctx_hbm.at[r], rep.at[k], csem).start()
                return carry

            lax.fori_loop(0, _CH, fill, 0)

            def fill_drain(k, carry):
                pltpu.make_async_copy(ctx_hbm.at[0], rep.at[0], csem).wait()
                return carry

            lax.fori_loop(0, _CH, fill_drain, 0)
            for j in range(_NCHUNK):
                pltpu.make_async_copy(
                    rep, out_hbm.at[r + 1, pl.ds(j * _CH, _CH)], csem).start()
            run_pipeline(16 * _N_A + (wid - 16) * _N_B, _N_B)
            for j in range(_NCHUNK):
                pltpu.make_async_copy(
                    rep, out_hbm.at[1, pl.ds(0, _CH)], csem).wait()

    return prompts_kernel


def kernel(ctx, token_prefix, token_suffix):
    out_t = _build_sc_kernel()(
        ctx,
        token_prefix.reshape(_N_CLS, _DIM),
        token_suffix.transpose(1, 0, 2),
    )
    return out_t.transpose(1, 0, 2)
